# Initial kernel scaffold; baseline (speedup 1.0000x reference)
import jax
import jax.numpy as jnp
from jax.experimental import pallas as pl


def kernel(x, W, b):
    raise NotImplementedError("write your pallas kernel here")


# speedup vs baseline: 1.1727x; 1.1727x over previous
import functools

import jax
import jax.numpy as jnp
from jax import lax
from jax.experimental import pallas as pl
from jax.experimental.pallas import tpu as pltpu

B, T, D = 4, 8192, 768
K = max(1, T // 10)
EMPH = 1.5
TT = 1024
NT = T // TT


def _k1_body(x_ref, w_ref, b_ref, u_ref, s1_ref):
    t = pl.program_id(1)
    xb = x_ref[0]
    wv = w_ref[...]
    e = lax.dot_general(wv, xb, (((1,), (1,)), ((), ())),
                        preferred_element_type=jnp.float32)
    u = jnp.exp(jnp.tanh(e + b_ref[0]))
    u_ref[0] = u
    s1 = lax.dot_general(u, xb, (((1,), (0,)), ((), ())),
                         preferred_element_type=jnp.float32)

    @pl.when(t == 0)
    def _():
        s1_ref[0] = s1

    @pl.when(t != 0)
    def _():
        s1_ref[0] = s1_ref[0] + s1


def _k1(x, wrow, bvec):
    return pl.pallas_call(
        _k1_body,
        grid=(B, NT),
        in_specs=[
            pl.BlockSpec((1, TT, D), lambda b, t: (b, t, 0)),
            pl.BlockSpec((1, D), lambda b, t: (0, 0)),
            pl.BlockSpec(memory_space=pltpu.SMEM),
        ],
        out_specs=[
            pl.BlockSpec((1, 1, TT), lambda b, t: (b, 0, t)),
            pl.BlockSpec((1, 1, D), lambda b, t: (b, 0, 0)),
        ],
        out_shape=[
            jax.ShapeDtypeStruct((B, 1, T), jnp.float32),
            jax.ShapeDtypeStruct((B, 1, D), jnp.float32),
        ],
    )(x, wrow, bvec)


def _k2_body(u_ref, stats_ref):
    for b in range(B):
        row = u_ref[b]
        bits = lax.bitcast_convert_type(row, jnp.int32)
        z = jnp.sum(row)

        def step(_, lohi):
            lo, hi = lohi
            mid = lo + (hi - lo) // 2
            c = jnp.sum(jnp.where(bits >= mid, 1, 0))
            big = c >= K
            return jnp.where(big, mid, lo), jnp.where(big, hi, mid)

        lo0 = jnp.int32(0)
        hi0 = jnp.int32(0x7F800000)
        lo, _ = lax.fori_loop(0, 31, step, (lo0, hi0))
        stats_ref[0, b] = lax.bitcast_convert_type(lo, jnp.float32)
        stats_ref[1, b] = z


def _k2(u):
    return pl.pallas_call(
        _k2_body,
        in_specs=[pl.BlockSpec(memory_space=pltpu.VMEM)],
        out_specs=pl.BlockSpec(memory_space=pltpu.SMEM),
        out_shape=jax.ShapeDtypeStruct((2, B), jnp.float32),
    )(u)


def _k3_body(stats_ref, x_ref, u_ref, s2_ref):
    b = pl.program_id(0)
    t = pl.program_id(1)
    th = stats_ref[0, b]
    u = u_ref[0]
    um = jnp.where(u >= th, u, 0.0)
    s2 = lax.dot_general(um, x_ref[0], (((1,), (0,)), ((), ())),
                         preferred_element_type=jnp.float32)

    @pl.when(t == 0)
    def _():
        s2_ref[0] = s2

    @pl.when(t != 0)
    def _():
        s2_ref[0] = s2_ref[0] + s2


def _k3_tc(stats, x, u):
    return pl.pallas_call(
        _k3_body,
        grid=(B, NT),
        in_specs=[
            pl.BlockSpec(memory_space=pltpu.SMEM),
            pl.BlockSpec((1, TT, D), lambda b, t: (b, t, 0)),
            pl.BlockSpec((1, 1, TT), lambda b, t: (b, 0, t)),
        ],
        out_specs=pl.BlockSpec((1, 1, D), lambda b, t: (b, 0, 0)),
        out_shape=jax.ShapeDtypeStruct((B, 1, D), jnp.float32),
    )(stats, x, u)


def _k4_body(stats_ref, s1_ref, s2_ref, out_ref):
    for b in range(B):
        z = stats_ref[1, b]
        out_ref[b] = (s1_ref[b] + (EMPH - 1.0) * s2_ref[b]) / z


def _k4(stats, s1, s2):
    return pl.pallas_call(
        _k4_body,
        in_specs=[
            pl.BlockSpec(memory_space=pltpu.SMEM),
            pl.BlockSpec(memory_space=pltpu.VMEM),
            pl.BlockSpec(memory_space=pltpu.VMEM),
        ],
        out_shape=jax.ShapeDtypeStruct((B, 1, D), jnp.float32),
    )(stats, s1, s2)


def kernel(x, W, b):
    wrow = W.reshape(1, D)
    u, s1 = _k1(x, wrow, b)
    stats = _k2(u)
    s2 = _k3_tc(stats, x, u)
    return _k4(stats, s1, s2)
